# Initial kernel scaffold; baseline (speedup 1.0000x reference)
import jax
import jax.numpy as jnp
from jax.experimental import pallas as pl


def kernel(z, dictionary):
    raise NotImplementedError("write your pallas kernel here")


# speedup vs baseline: 200.4582x; 200.4582x over previous
import jax
import jax.numpy as jnp
from jax.experimental import pallas as pl

NUM_ATOMS = 512
DIM = 64
K = 5
EPS = 1e-10
DIAG_EPS = 1e-5
BLK = 512

_HI = jax.lax.Precision.HIGHEST


def _bf(x):
    return x.astype(jnp.bfloat16).astype(jnp.float32)


def _omp_body(xT_ref, d_ref, out_ref):
    D = d_ref[...]
    nrm = jnp.sqrt(jnp.sum(D * D, axis=0, keepdims=True))
    D = D / (nrm + EPS)
    Dbf = D.astype(jnp.bfloat16)
    X = xT_ref[...]
    Xbf = _bf(X)

    row = jax.lax.broadcasted_iota(jnp.int32, (NUM_ATOMS, BLK), 0)

    atoms = []
    atomsbf = []
    bs = []
    idxs = []
    G = {}
    recon = jnp.zeros_like(X)

    for t in range(K):
        r = X - recon
        corr = jax.lax.dot_general(
            Dbf, r.astype(jnp.bfloat16), (((0,), (0,)), ((), ())),
            preferred_element_type=jnp.float32,
        )
        a = jnp.abs(corr)
        for idx in idxs:
            a = jnp.where(row == idx, -1.0, a)
        m = jnp.max(a, axis=0, keepdims=True)
        new = jnp.min(
            jnp.where(a == m, row, NUM_ATOMS), axis=0, keepdims=True
        ).astype(jnp.int32)
        onehot = (row == new).astype(jnp.float32)
        at = jax.lax.dot_general(
            D, onehot, (((1,), (0,)), ((), ())),
            preferred_element_type=jnp.float32,
            precision=_HI,
        )
        atbf = _bf(at)
        for j, ajbf in enumerate(atomsbf):
            G[(t, j)] = jnp.sum(atbf * ajbf, axis=0, keepdims=True)
        G[(t, t)] = jnp.sum(atbf * atbf, axis=0, keepdims=True) + DIAG_EPS
        atoms.append(at)
        atomsbf.append(atbf)
        bs.append(jnp.sum(atbf * Xbf, axis=0, keepdims=True))
        idxs.append(new)

        n = t + 1
        if n == 1:
            g00 = jnp.sum(at * at, axis=0, keepdims=True) + DIAG_EPS
            b0 = jnp.sum(at * X, axis=0, keepdims=True)
            w = [b0 / g00]
        else:
            L = [[None] * n for _ in range(n)]
            inv_d = [None] * n
            for i in range(n):
                for j in range(i):
                    s = G[(i, j)]
                    for kk in range(j):
                        s = s - L[i][kk] * L[j][kk]
                    L[i][j] = s * inv_d[j]
                s = G[(i, i)]
                for kk in range(i):
                    s = s - L[i][kk] * L[i][kk]
                L[i][i] = jnp.sqrt(s)
                inv_d[i] = 1.0 / L[i][i]
            y = []
            for i in range(n):
                s = bs[i]
                for j in range(i):
                    s = s - L[i][j] * y[j]
                y.append(s * inv_d[i])
            w = [None] * n
            for i in reversed(range(n)):
                s = y[i]
                for j in range(i + 1, n):
                    s = s - L[j][i] * w[j]
                w[i] = s * inv_d[i]

        recon = jnp.zeros_like(X)
        for wt, atbf_t in zip(w, atomsbf):
            recon = recon + _bf(wt) * atbf_t

    out_ref[...] = recon


def _omp_pallas(xT, dictionary):
    C, S = xT.shape
    return pl.pallas_call(
        _omp_body,
        grid=(S // BLK,),
        in_specs=[
            pl.BlockSpec((C, BLK), lambda i: (0, i)),
            pl.BlockSpec((DIM, NUM_ATOMS), lambda i: (0, 0)),
        ],
        out_specs=pl.BlockSpec((C, BLK), lambda i: (0, i)),
        out_shape=jax.ShapeDtypeStruct((C, S), jnp.float32),
    )(xT, dictionary)


@jax.jit
def kernel(z, dictionary):
    B, C, H, W = z.shape
    xT = jnp.transpose(z, (1, 0, 2, 3)).reshape(C, B * H * W)
    qT = _omp_pallas(xT, dictionary)
    q = jnp.transpose(qT.reshape(C, B, H, W), (1, 0, 2, 3))
    return z + jax.lax.stop_gradient(q - z)


# speedup vs baseline: 287.2341x; 1.4329x over previous
import jax
import jax.numpy as jnp
from jax.experimental import pallas as pl

NUM_ATOMS = 512
DIM = 64
K = 5
EPS = 1e-10
DIAG_EPS = 1e-5
BLK = 512


def _bf(x):
    return x.astype(jnp.bfloat16).astype(jnp.float32)


def _omp_body(xT_ref, d_ref, out_ref):
    D = d_ref[...]
    nrm = jnp.sqrt(jnp.sum(D * D, axis=0, keepdims=True))
    D = D / (nrm + EPS)
    Dhi = D.astype(jnp.bfloat16)
    X = xT_ref[...]
    Xbf = _bf(X)

    row = jax.lax.broadcasted_iota(jnp.int32, (NUM_ATOMS, BLK), 0)
    sel = jnp.zeros((NUM_ATOMS, BLK), jnp.bool_)

    atomsbf = []
    bs = []
    G = {}
    recon = jnp.zeros_like(X)

    def gather_dot(dpart, onehot):
        return jax.lax.dot_general(
            dpart, onehot, (((1,), (0,)), ((), ())),
            preferred_element_type=jnp.float32,
        )

    for t in range(K):
        r = X - recon
        corr = jax.lax.dot_general(
            Dhi, r.astype(jnp.bfloat16), (((0,), (0,)), ((), ())),
            preferred_element_type=jnp.float32,
        )
        a = jnp.where(sel, -1.0, jnp.abs(corr))
        m = jnp.max(a, axis=0, keepdims=True)
        new = jnp.min(
            jnp.where(a == m, row, NUM_ATOMS), axis=0, keepdims=True
        ).astype(jnp.int32)
        oh = row == new
        if t + 1 < K:
            sel = sel | oh
        onehot = oh.astype(jnp.bfloat16)
        atbf = gather_dot(Dhi, onehot)
        if t == 0:
            at = jax.lax.dot_general(
                D, oh.astype(jnp.float32), (((1,), (0,)), ((), ())),
                preferred_element_type=jnp.float32,
                precision=jax.lax.Precision.HIGHEST,
            )
            g00 = jnp.sum(at * at, axis=0, keepdims=True) + DIAG_EPS
            b0 = jnp.sum(at * X, axis=0, keepdims=True)
        for j, ajbf in enumerate(atomsbf):
            G[(t, j)] = jnp.sum(atbf * ajbf, axis=0, keepdims=True)
        G[(t, t)] = jnp.sum(atbf * atbf, axis=0, keepdims=True) + DIAG_EPS
        atomsbf.append(atbf)
        bs.append(jnp.sum(atbf * Xbf, axis=0, keepdims=True))

        n = t + 1
        if n == 1:
            w = [b0 / g00]
        else:
            L = [[None] * n for _ in range(n)]
            inv_d = [None] * n
            for i in range(n):
                for j in range(i):
                    s = G[(i, j)]
                    for kk in range(j):
                        s = s - L[i][kk] * L[j][kk]
                    L[i][j] = s * inv_d[j]
                s = G[(i, i)]
                for kk in range(i):
                    s = s - L[i][kk] * L[i][kk]
                L[i][i] = jnp.sqrt(s)
                inv_d[i] = 1.0 / L[i][i]
            y = []
            for i in range(n):
                s = bs[i]
                for j in range(i):
                    s = s - L[i][j] * y[j]
                y.append(s * inv_d[i])
            w = [None] * n
            for i in reversed(range(n)):
                s = y[i]
                for j in range(i + 1, n):
                    s = s - L[j][i] * w[j]
                w[i] = s * inv_d[i]

        recon = jnp.zeros_like(X)
        for wt, atbf_t in zip(w, atomsbf):
            recon = recon + _bf(wt) * atbf_t

    out_ref[...] = recon


def _omp_pallas(xT, dictionary):
    C, S = xT.shape
    return pl.pallas_call(
        _omp_body,
        grid=(S // BLK,),
        in_specs=[
            pl.BlockSpec((C, BLK), lambda i: (0, i)),
            pl.BlockSpec((DIM, NUM_ATOMS), lambda i: (0, 0)),
        ],
        out_specs=pl.BlockSpec((C, BLK), lambda i: (0, i)),
        out_shape=jax.ShapeDtypeStruct((C, S), jnp.float32),
    )(xT, dictionary)


@jax.jit
def kernel(z, dictionary):
    B, C, H, W = z.shape
    xT = jnp.transpose(z, (1, 0, 2, 3)).reshape(C, B * H * W)
    qT = _omp_pallas(xT, dictionary)
    q = jnp.transpose(qT.reshape(C, B, H, W), (1, 0, 2, 3))
    return z + jax.lax.stop_gradient(q - z)


# speedup vs baseline: 305.5860x; 1.0639x over previous
import jax
import jax.numpy as jnp
from jax.experimental import pallas as pl

NUM_ATOMS = 512
DIM = 64
K = 5
EPS = 1e-10
DIAG_EPS = 1e-5
BLK = 512


def _bf(x):
    return x.astype(jnp.bfloat16).astype(jnp.float32)


def _omp_body(xT_ref, d_ref, out_ref):
    D = d_ref[...]
    nrm = jnp.sqrt(jnp.sum(D * D, axis=0, keepdims=True))
    D = D / (nrm + EPS)
    Dhi = D.astype(jnp.bfloat16)
    X = xT_ref[0]
    Xbf = _bf(X)

    row = jax.lax.broadcasted_iota(jnp.int32, (NUM_ATOMS, BLK), 0)
    sel = jnp.zeros((NUM_ATOMS, BLK), jnp.bool_)

    atomsbf = []
    bs = []
    G = {}
    recon = jnp.zeros_like(X)

    def gather_dot(dpart, onehot):
        return jax.lax.dot_general(
            dpart, onehot, (((1,), (0,)), ((), ())),
            preferred_element_type=jnp.float32,
        )

    for t in range(K):
        r = X - recon
        corr = jax.lax.dot_general(
            Dhi, r.astype(jnp.bfloat16), (((0,), (0,)), ((), ())),
            preferred_element_type=jnp.float32,
        )
        a = jnp.where(sel, -1.0, jnp.abs(corr))
        m = jnp.max(a, axis=0, keepdims=True)
        new = jnp.min(
            jnp.where(a == m, row, NUM_ATOMS), axis=0, keepdims=True
        ).astype(jnp.int32)
        oh = row == new
        if t + 1 < K:
            sel = sel | oh
        onehot = oh.astype(jnp.bfloat16)
        atbf = gather_dot(Dhi, onehot)
        if t == 0:
            at = jax.lax.dot_general(
                D, oh.astype(jnp.float32), (((1,), (0,)), ((), ())),
                preferred_element_type=jnp.float32,
                precision=jax.lax.Precision.HIGHEST,
            )
            g00 = jnp.sum(at * at, axis=0, keepdims=True) + DIAG_EPS
            b0 = jnp.sum(at * X, axis=0, keepdims=True)
        for j, ajbf in enumerate(atomsbf):
            G[(t, j)] = jnp.sum(atbf * ajbf, axis=0, keepdims=True)
        G[(t, t)] = jnp.sum(atbf * atbf, axis=0, keepdims=True) + DIAG_EPS
        atomsbf.append(atbf)
        bs.append(jnp.sum(atbf * Xbf, axis=0, keepdims=True))

        n = t + 1
        if n == 1:
            w = [b0 / g00]
        else:
            L = [[None] * n for _ in range(n)]
            inv_d = [None] * n
            for i in range(n):
                for j in range(i):
                    s = G[(i, j)]
                    for kk in range(j):
                        s = s - L[i][kk] * L[j][kk]
                    L[i][j] = s * inv_d[j]
                s = G[(i, i)]
                for kk in range(i):
                    s = s - L[i][kk] * L[i][kk]
                L[i][i] = jnp.sqrt(s)
                inv_d[i] = 1.0 / L[i][i]
            y = []
            for i in range(n):
                s = bs[i]
                for j in range(i):
                    s = s - L[i][j] * y[j]
                y.append(s * inv_d[i])
            w = [None] * n
            for i in reversed(range(n)):
                s = y[i]
                for j in range(i + 1, n):
                    s = s - L[j][i] * w[j]
                w[i] = s * inv_d[i]

        recon = jnp.zeros_like(X)
        for wt, atbf_t in zip(w, atomsbf):
            recon = recon + _bf(wt) * atbf_t

    out_ref[0] = recon


def _omp_pallas(zf, dictionary):
    B, C, S = zf.shape
    nb = S // BLK
    return pl.pallas_call(
        _omp_body,
        grid=(B * nb,),
        in_specs=[
            pl.BlockSpec((1, C, BLK), lambda i: (i // nb, 0, i % nb)),
            pl.BlockSpec((DIM, NUM_ATOMS), lambda i: (0, 0)),
        ],
        out_specs=pl.BlockSpec((1, C, BLK), lambda i: (i // nb, 0, i % nb)),
        out_shape=jax.ShapeDtypeStruct((B, C, S), jnp.float32),
    )(zf, dictionary)


@jax.jit
def kernel(z, dictionary):
    B, C, H, W = z.shape
    q = _omp_pallas(z.reshape(B, C, H * W), dictionary).reshape(B, C, H, W)
    return z + jax.lax.stop_gradient(q - z)


# speedup vs baseline: 389.9717x; 1.2761x over previous
import jax
import jax.numpy as jnp
from jax.experimental import pallas as pl

NUM_ATOMS = 512
DIM = 64
K = 5
EPS = 1e-10
DIAG_EPS = 1e-5
BLK = 1024


def _bf(x):
    return x.astype(jnp.bfloat16).astype(jnp.float32)


def _omp_body(xT_ref, d_ref, out_ref):
    D = d_ref[...]
    nrm = jnp.sqrt(jnp.sum(D * D, axis=0, keepdims=True))
    D = D / (nrm + EPS)
    Dhi = D.astype(jnp.bfloat16)
    X = xT_ref[0]
    Xbf = _bf(X)

    row = jax.lax.broadcasted_iota(jnp.int32, (NUM_ATOMS, BLK), 0)
    sel = jnp.zeros((NUM_ATOMS, BLK), jnp.bool_)

    atomsbf = []
    bs = []
    G = {}
    recon = jnp.zeros_like(X)

    def gather_dot(dpart, onehot):
        return jax.lax.dot_general(
            dpart, onehot, (((1,), (0,)), ((), ())),
            preferred_element_type=jnp.float32,
        )

    for t in range(K):
        r = X - recon
        corr = jax.lax.dot_general(
            Dhi, r.astype(jnp.bfloat16), (((0,), (0,)), ((), ())),
            preferred_element_type=jnp.float32,
        )
        a = jnp.where(sel, -1.0, jnp.abs(corr))
        m = jnp.max(a, axis=0, keepdims=True)
        new = jnp.min(
            jnp.where(a == m, row, NUM_ATOMS), axis=0, keepdims=True
        ).astype(jnp.int32)
        oh = row == new
        if t + 1 < K:
            sel = sel | oh
        onehot = oh.astype(jnp.bfloat16)
        atbf = gather_dot(Dhi, onehot)
        if t == 0:
            at = jax.lax.dot_general(
                D, oh.astype(jnp.float32), (((1,), (0,)), ((), ())),
                preferred_element_type=jnp.float32,
                precision=jax.lax.Precision.HIGHEST,
            )
            g00 = jnp.sum(at * at, axis=0, keepdims=True) + DIAG_EPS
            b0 = jnp.sum(at * X, axis=0, keepdims=True)
        for j, ajbf in enumerate(atomsbf):
            G[(t, j)] = jnp.sum(atbf * ajbf, axis=0, keepdims=True)
        G[(t, t)] = jnp.sum(atbf * atbf, axis=0, keepdims=True) + DIAG_EPS
        atomsbf.append(atbf)
        bs.append(jnp.sum(atbf * Xbf, axis=0, keepdims=True))

        n = t + 1
        if n == 1:
            w = [b0 / g00]
        else:
            L = [[None] * n for _ in range(n)]
            inv_d = [None] * n
            for i in range(n):
                for j in range(i):
                    s = G[(i, j)]
                    for kk in range(j):
                        s = s - L[i][kk] * L[j][kk]
                    L[i][j] = s * inv_d[j]
                s = G[(i, i)]
                for kk in range(i):
                    s = s - L[i][kk] * L[i][kk]
                L[i][i] = jnp.sqrt(s)
                inv_d[i] = 1.0 / L[i][i]
            y = []
            for i in range(n):
                s = bs[i]
                for j in range(i):
                    s = s - L[i][j] * y[j]
                y.append(s * inv_d[i])
            w = [None] * n
            for i in reversed(range(n)):
                s = y[i]
                for j in range(i + 1, n):
                    s = s - L[j][i] * w[j]
                w[i] = s * inv_d[i]

        recon = jnp.zeros_like(X)
        for wt, atbf_t in zip(w, atomsbf):
            recon = recon + _bf(wt) * atbf_t

    out_ref[0] = recon


def _omp_pallas(zf, dictionary):
    B, C, S = zf.shape
    nb = S // BLK
    return pl.pallas_call(
        _omp_body,
        grid=(B * nb,),
        in_specs=[
            pl.BlockSpec((1, C, BLK), lambda i: (i // nb, 0, i % nb)),
            pl.BlockSpec((DIM, NUM_ATOMS), lambda i: (0, 0)),
        ],
        out_specs=pl.BlockSpec((1, C, BLK), lambda i: (i // nb, 0, i % nb)),
        out_shape=jax.ShapeDtypeStruct((B, C, S), jnp.float32),
    )(zf, dictionary)


@jax.jit
def kernel(z, dictionary):
    B, C, H, W = z.shape
    q = _omp_pallas(z.reshape(B, C, H * W), dictionary).reshape(B, C, H, W)
    return z + jax.lax.stop_gradient(q - z)


# speedup vs baseline: 416.6849x; 1.0685x over previous
import jax
import jax.numpy as jnp
from jax.experimental import pallas as pl

NUM_ATOMS = 512
DIM = 64
K = 5
EPS = 1e-10
DIAG_EPS = 1e-5
BLK = 2048


def _bf(x):
    return x.astype(jnp.bfloat16).astype(jnp.float32)


def _omp_body(xT_ref, d_ref, out_ref):
    D = d_ref[...]
    nrm = jnp.sqrt(jnp.sum(D * D, axis=0, keepdims=True))
    D = D / (nrm + EPS)
    Dhi = D.astype(jnp.bfloat16)
    X = xT_ref[0]
    Xbf = _bf(X)

    row = jax.lax.broadcasted_iota(jnp.int32, (NUM_ATOMS, BLK), 0)
    sel = jnp.zeros((NUM_ATOMS, BLK), jnp.bool_)

    atomsbf = []
    bs = []
    G = {}
    recon = jnp.zeros_like(X)

    def gather_dot(dpart, onehot):
        return jax.lax.dot_general(
            dpart, onehot, (((1,), (0,)), ((), ())),
            preferred_element_type=jnp.float32,
        )

    for t in range(K):
        r = X - recon
        corr = jax.lax.dot_general(
            Dhi, r.astype(jnp.bfloat16), (((0,), (0,)), ((), ())),
            preferred_element_type=jnp.float32,
        )
        a = jnp.where(sel, -1.0, jnp.abs(corr))
        m = jnp.max(a, axis=0, keepdims=True)
        new = jnp.min(
            jnp.where(a == m, row, NUM_ATOMS), axis=0, keepdims=True
        ).astype(jnp.int32)
        oh = row == new
        if t + 1 < K:
            sel = sel | oh
        onehot = oh.astype(jnp.bfloat16)
        atbf = gather_dot(Dhi, onehot)
        if t == 0:
            at = jax.lax.dot_general(
                D, oh.astype(jnp.float32), (((1,), (0,)), ((), ())),
                preferred_element_type=jnp.float32,
                precision=jax.lax.Precision.HIGHEST,
            )
            g00 = jnp.sum(at * at, axis=0, keepdims=True) + DIAG_EPS
            b0 = jnp.sum(at * X, axis=0, keepdims=True)
        for j, ajbf in enumerate(atomsbf):
            G[(t, j)] = jnp.sum(atbf * ajbf, axis=0, keepdims=True)
        G[(t, t)] = jnp.sum(atbf * atbf, axis=0, keepdims=True) + DIAG_EPS
        atomsbf.append(atbf)
        bs.append(jnp.sum(atbf * Xbf, axis=0, keepdims=True))

        n = t + 1
        if n == 1:
            w = [b0 / g00]
        else:
            L = [[None] * n for _ in range(n)]
            inv_d = [None] * n
            for i in range(n):
                for j in range(i):
                    s = G[(i, j)]
                    for kk in range(j):
                        s = s - L[i][kk] * L[j][kk]
                    L[i][j] = s * inv_d[j]
                s = G[(i, i)]
                for kk in range(i):
                    s = s - L[i][kk] * L[i][kk]
                L[i][i] = jnp.sqrt(s)
                inv_d[i] = 1.0 / L[i][i]
            y = []
            for i in range(n):
                s = bs[i]
                for j in range(i):
                    s = s - L[i][j] * y[j]
                y.append(s * inv_d[i])
            w = [None] * n
            for i in reversed(range(n)):
                s = y[i]
                for j in range(i + 1, n):
                    s = s - L[j][i] * w[j]
                w[i] = s * inv_d[i]

        recon = jnp.zeros_like(X)
        for wt, atbf_t in zip(w, atomsbf):
            recon = recon + _bf(wt) * atbf_t

    out_ref[0] = recon


def _omp_pallas(zf, dictionary):
    B, C, S = zf.shape
    nb = S // BLK
    return pl.pallas_call(
        _omp_body,
        grid=(B * nb,),
        in_specs=[
            pl.BlockSpec((1, C, BLK), lambda i: (i // nb, 0, i % nb)),
            pl.BlockSpec((DIM, NUM_ATOMS), lambda i: (0, 0)),
        ],
        out_specs=pl.BlockSpec((1, C, BLK), lambda i: (i // nb, 0, i % nb)),
        out_shape=jax.ShapeDtypeStruct((B, C, S), jnp.float32),
    )(zf, dictionary)


@jax.jit
def kernel(z, dictionary):
    B, C, H, W = z.shape
    q = _omp_pallas(z.reshape(B, C, H * W), dictionary).reshape(B, C, H, W)
    return z + jax.lax.stop_gradient(q - z)
